# baseline (speedup 1.0000x reference)
import jax
import jax.numpy as jnp
from jax import lax
from jax.experimental import pallas as pl
from jax.experimental.pallas import tpu as pltpu
from jax.experimental.pallas import tpu_sc as plsc

_NW = 32
_CHUNK = 128
_CHUNKS_PER_W = 64
_D = 128


def _embed_body(table_hbm, idx_hbm, out_hbm, table_v, idx_v, r0, r1, r2, r3, gsem, wsem):
    c = lax.axis_index("c")
    s = lax.axis_index("s")
    wid = s * 2 + c
    pltpu.sync_copy(table_hbm, table_v)
    pltpu.sync_copy(idx_hbm.at[wid], idx_v)
    bufs = (r0, r1, r2, r3)

    def gather(j, slot):
        pltpu.async_copy(table_v.at[idx_v.at[j]], bufs[slot], gsem.at[slot])

    def wait_gather(j, slot):
        pltpu.make_async_copy(table_v.at[idx_v.at[j]], bufs[slot], gsem.at[slot]).wait()

    def write(j, slot):
        pltpu.async_copy(bufs[slot], out_hbm.at[wid, j], wsem.at[slot])

    def wait_write(j, slot):
        pltpu.make_async_copy(bufs[slot], out_hbm.at[wid, j], wsem.at[slot]).wait()

    gather(0, 0)
    gather(1, 1)
    wait_gather(0, 0)
    write(0, 0)
    gather(2, 2)
    wait_gather(1, 1)
    write(1, 1)
    gather(3, 3)

    def outer(i, carry):
        j0 = i * 4 + 2
        for b in range(4):
            j = j0 + b
            slot = (2 + b) % 4
            wait_gather(j, slot)
            write(j, slot)
            wait_write(j - 2, b)
            gather(j + 2, b)
        return carry

    lax.fori_loop(0, 15, outer, 0)

    wait_gather(62, 2)
    write(62, 2)
    wait_write(60, 0)
    wait_gather(63, 3)
    write(63, 3)
    wait_write(61, 1)
    wait_write(62, 2)
    wait_write(63, 3)


def kernel(species, W):
    idx = species.reshape(_NW, _CHUNKS_PER_W, _CHUNK)
    mesh = plsc.VectorSubcoreMesh(core_axis_name="c", subcore_axis_name="s")
    k = pl.kernel(
        _embed_body,
        out_type=jax.ShapeDtypeStruct((_NW, _CHUNKS_PER_W, _CHUNK, _D), jnp.float32),
        mesh=mesh,
        scratch_types=[
            pltpu.VMEM_SHARED((92, _D), jnp.float32),
            pltpu.VMEM((_CHUNKS_PER_W, _CHUNK), jnp.int32),
            pltpu.VMEM((_CHUNK, _D), jnp.float32),
            pltpu.VMEM((_CHUNK, _D), jnp.float32),
            pltpu.VMEM((_CHUNK, _D), jnp.float32),
            pltpu.VMEM((_CHUNK, _D), jnp.float32),
            pltpu.SemaphoreType.DMA((4,)),
            pltpu.SemaphoreType.DMA((4,)),
        ],
    )
    out = k(W, idx)
    b, g = species.shape[0], species.shape[1]
    return out.reshape(b, g, g, g, _D)
